# Initial kernel scaffold; baseline (speedup 1.0000x reference)
import jax
import jax.numpy as jnp
from jax.experimental import pallas as pl


def kernel(x, edge_index, edge_attr, batch, W_e1a, b_e1a, W_e1b, b_e1b, root1, bias1, g1, be1, W_e2a, b_e2a, W_e2b, b_e2b, root2, bias2, g2, be2, Wl, bl):
    raise NotImplementedError("write your pallas kernel here")


# speedup vs baseline: 1.5148x; 1.5148x over previous
import functools

import jax
import jax.numpy as jnp
from jax import lax
from jax.experimental import pallas as pl
from jax.experimental.pallas import tpu as pltpu
from jax.experimental.pallas import tpu_sc as plsc

N = 10000
E = 160000
IN = 128
ED = 16
H = 16
OUT = 128
NG = 64
K = 32
CW = 640

_info = plsc.get_sparse_core_info()
_NC = _info.num_cores
_NS = _info.num_subcores
_NW = _NC * _NS
_B = 40
_EW = E // _NW
_ITERS = _EW // _B


def _mm_body(a_ref, w_ref, o_ref):
    o_ref[...] = jnp.dot(a_ref[...], w_ref[...],
                         preferred_element_type=jnp.float32)


def _matmul(a, w, block_rows):
    m, k = a.shape
    _, c = w.shape
    return pl.pallas_call(
        _mm_body,
        grid=(m // block_rows,),
        in_specs=[pl.BlockSpec((block_rows, k), lambda i: (i, 0)),
                  pl.BlockSpec((k, c), lambda i: (0, 0))],
        out_specs=pl.BlockSpec((block_rows, c), lambda i: (i, 0)),
        out_shape=jax.ShapeDtypeStruct((m, c), jnp.float32),
    )(a, w)


def _h_body(ea_ref, w1_ref, b1_ref, w2_ref, b2_ref, h1_ref, h2_ref):
    ea = ea_ref[...]
    h1_ref[...] = jnp.maximum(
        jnp.dot(ea, w1_ref[...], preferred_element_type=jnp.float32)
        + b1_ref[...], 0.0)
    h2_ref[...] = jnp.maximum(
        jnp.dot(ea, w2_ref[...], preferred_element_type=jnp.float32)
        + b2_ref[...], 0.0)


def _edge_mlps(ea, w1, b1, w2, b2):
    br = 4000
    return pl.pallas_call(
        _h_body,
        grid=(E // br,),
        in_specs=[pl.BlockSpec((br, ED), lambda i: (i, 0)),
                  pl.BlockSpec((ED, K), lambda i: (0, 0)),
                  pl.BlockSpec((1, K), lambda i: (0, 0)),
                  pl.BlockSpec((ED, K), lambda i: (0, 0)),
                  pl.BlockSpec((1, K), lambda i: (0, 0))],
        out_specs=[pl.BlockSpec((br, K), lambda i: (i, 0)),
                   pl.BlockSpec((br, K), lambda i: (i, 0))],
        out_shape=[jax.ShapeDtypeStruct((E, K), jnp.float32),
                   jax.ShapeDtypeStruct((E, K), jnp.float32)],
    )(ea, w1, b1.reshape(1, K), w2, b2.reshape(1, K))


def _node_body(sp_ref, xr_ref, bias_ref, g_ref, be_ref, o_ref):
    sp = sp_ref[0] + sp_ref[1]
    s = sp[:, 0:H]
    c = jnp.maximum(sp[:, H:H + 1], 1.0)
    z = xr_ref[...] + s / c + bias_ref[...]
    mu = jnp.mean(z, axis=0, keepdims=True)
    var = jnp.mean((z - mu) * (z - mu), axis=0, keepdims=True)
    o_ref[...] = jnp.maximum(
        (z - mu) * lax.rsqrt(var + 1e-5) * g_ref[...] + be_ref[...], 0.0)


def _node_update(sparts, xr, bias, g, be):
    return pl.pallas_call(
        _node_body,
        in_specs=[pl.BlockSpec((_NC, N, SW), lambda: (0, 0, 0)),
                  pl.BlockSpec((N, H), lambda: (0, 0)),
                  pl.BlockSpec((1, H), lambda: (0, 0)),
                  pl.BlockSpec((1, H), lambda: (0, 0)),
                  pl.BlockSpec((1, H), lambda: (0, 0))],
        out_specs=pl.BlockSpec((N, H), lambda: (0, 0)),
        out_shape=jax.ShapeDtypeStruct((N, H), jnp.float32),
    )(sparts, xr, bias.reshape(1, H), g.reshape(1, H),
      be.reshape(1, H))


def _pool_body(x2_ref, batch_ref, wl_ref, bl_ref, o_ref):
    x2 = x2_ref[...]
    b = batch_ref[...]
    gids = lax.broadcasted_iota(jnp.int32, (1, NG), 1)
    onehot = (b == gids).astype(jnp.float32)
    sums = lax.dot_general(onehot, x2, (((0,), (0,)), ((), ())),
                           preferred_element_type=jnp.float32)
    cnt = jnp.maximum(jnp.sum(onehot, axis=0), 1.0)
    pooled = sums / cnt[:, None]
    o_ref[...] = jnp.dot(pooled, wl_ref[...],
                         preferred_element_type=jnp.float32) + bl_ref[...]


def _pool_project(x2, batch, wl, bl):
    return pl.pallas_call(
        _pool_body,
        in_specs=[pl.BlockSpec((N, H), lambda: (0, 0)),
                  pl.BlockSpec((N, 1), lambda: (0, 0)),
                  pl.BlockSpec((H, OUT), lambda: (0, 0)),
                  pl.BlockSpec((1, OUT), lambda: (0, 0))],
        out_specs=pl.BlockSpec((NG, OUT), lambda: (0, 0)),
        out_shape=jax.ShapeDtypeStruct((NG, OUT), jnp.float32),
    )(x2, batch.reshape(N, 1), wl, bl.reshape(1, OUT))


SW = 128


def _sc_body(pcat_hbm, h_hbm, src_hbm, dst_hbm, zeros_hbm,
             s_out,
             src_v, dst_v, rows_v, h_v, msg_v, s_sh, sem):
    cid = lax.axis_index("c")
    sid = lax.axis_index("s")
    wid = sid * _NC + cid

    @pl.when(sid == 0)
    def _():
        pltpu.sync_copy(zeros_hbm, s_sh)
    plsc.subcore_barrier()

    zero16 = jnp.zeros((16,), jnp.float32)
    cnt16 = jnp.where(lax.iota(jnp.int32, 16) == 0, 1.0, 0.0)

    def _init_rows(b, carry):
        msg_v[b, pl.ds(16, 16)] = cnt16
        for j in range(2, 8):
            msg_v[b, pl.ds(j * 16, 16)] = zero16
        return carry
    lax.fori_loop(0, _B, _init_rows, 0)

    base0 = wid * _EW

    def _iter(i, carry):
        base = base0 + i * _B
        pltpu.sync_copy(src_hbm.at[pl.ds(base, _B)], src_v)
        pltpu.sync_copy(dst_hbm.at[pl.ds(base, _B)], dst_v)
        pltpu.sync_copy(h_hbm.at[pl.ds(base, _B)], h_v)
        pltpu.async_copy(pcat_hbm.at[src_v], rows_v, sem).wait()

        def _edge(b, c2):
            ha = h_v[b, pl.ds(0, 16)]
            hb = h_v[b, pl.ds(16, 16)]
            acc = rows_v[b, pl.ds(512, 16)]
            for kk in range(16):
                acc = acc + ha[kk] * rows_v[b, pl.ds(kk * 16, 16)]
            for kk in range(16):
                acc = acc + hb[kk] * rows_v[b, pl.ds((16 + kk) * 16, 16)]
            msg_v[b, pl.ds(0, 16)] = acc
            return c2
        lax.fori_loop(0, _B, _edge, 0)

        pltpu.sync_copy(msg_v, s_sh.at[dst_v], add=True)
        return carry
    lax.fori_loop(0, _ITERS, _iter, 0)

    plsc.subcore_barrier()

    @pl.when(sid == 0)
    def _():
        pltpu.sync_copy(s_sh, s_out.at[cid])


_sc_edge = pl.kernel(
    _sc_body,
    out_type=jax.ShapeDtypeStruct((_NC, N, SW), jnp.float32),
    mesh=plsc.VectorSubcoreMesh(core_axis_name="c", subcore_axis_name="s"),
    scratch_types=[
        pltpu.VMEM((_B,), jnp.int32),
        pltpu.VMEM((_B,), jnp.int32),
        pltpu.VMEM((_B, CW), jnp.float32),
        pltpu.VMEM((_B, K), jnp.float32),
        pltpu.VMEM((_B, SW), jnp.float32),
        pltpu.VMEM_SHARED((N, SW), jnp.float32),
        pltpu.SemaphoreType.DMA,
    ],
)


def kernel(x, edge_index, edge_attr, batch,
           W_e1a, b_e1a, W_e1b, b_e1b, root1, bias1, g1, be1,
           W_e2a, b_e2a, W_e2b, b_e2b, root2, bias2, g2, be2, Wl, bl):
    src = edge_index[0]
    dst = edge_index[1]
    zeros = jnp.zeros((N, SW), jnp.float32)

    a1 = W_e1b.reshape(K, IN, H).transpose(1, 0, 2).reshape(IN, K * H)
    acat1 = jnp.concatenate(
        [a1, b_e1b.reshape(IN, H), root1, jnp.zeros((IN, 96), jnp.float32)],
        axis=1)
    a2 = W_e2b.reshape(K, H, H).transpose(1, 0, 2).reshape(H, K * H)
    acat2 = jnp.concatenate(
        [a2, b_e2b.reshape(H, H), root2, jnp.zeros((H, 96), jnp.float32)],
        axis=1)

    h1, h2 = _edge_mlps(edge_attr, W_e1a, b_e1a, W_e2a, b_e2a)

    full1 = _matmul(x, acat1, 1000)
    s1 = _sc_edge(full1, h1, src, dst, zeros)
    x1 = _node_update(s1, full1[:, 528:544], bias1, g1, be1)

    full2 = _matmul(x1, acat2, 1000)
    s2 = _sc_edge(full2, h2, src, dst, zeros)
    x2 = _node_update(s2, full2[:, 528:544], bias2, g2, be2)

    return _pool_project(x2, batch, Wl, bl)


# speedup vs baseline: 2.1049x; 1.3896x over previous
import functools

import jax
import jax.numpy as jnp
from jax import lax
from jax.experimental import pallas as pl
from jax.experimental.pallas import tpu as pltpu
from jax.experimental.pallas import tpu_sc as plsc

N = 10000
E = 160000
IN = 128
ED = 16
H = 16
OUT = 128
NG = 64
K = 32
CW = 640

_info = plsc.get_sparse_core_info()
_NC = _info.num_cores
_NS = _info.num_subcores
_NW = _NC * _NS
_B = 40
_EW = E // _NW
_ITERS = _EW // _B


def _mm_body(a_ref, w_ref, o_ref):
    o_ref[...] = jnp.dot(a_ref[...], w_ref[...],
                         preferred_element_type=jnp.float32)


def _matmul(a, w, block_rows):
    m, k = a.shape
    _, c = w.shape
    return pl.pallas_call(
        _mm_body,
        grid=(m // block_rows,),
        in_specs=[pl.BlockSpec((block_rows, k), lambda i: (i, 0)),
                  pl.BlockSpec((k, c), lambda i: (0, 0))],
        out_specs=pl.BlockSpec((block_rows, c), lambda i: (i, 0)),
        out_shape=jax.ShapeDtypeStruct((m, c), jnp.float32),
    )(a, w)


def _h_body(ea_ref, w1_ref, b1_ref, w2_ref, b2_ref, h1_ref, h2_ref):
    ea = ea_ref[...]
    h1_ref[...] = jnp.maximum(
        jnp.dot(ea, w1_ref[...], preferred_element_type=jnp.float32)
        + b1_ref[...], 0.0)
    h2_ref[...] = jnp.maximum(
        jnp.dot(ea, w2_ref[...], preferred_element_type=jnp.float32)
        + b2_ref[...], 0.0)


def _edge_mlps(ea, w1, b1, w2, b2):
    br = 4000
    return pl.pallas_call(
        _h_body,
        grid=(E // br,),
        in_specs=[pl.BlockSpec((br, ED), lambda i: (i, 0)),
                  pl.BlockSpec((ED, K), lambda i: (0, 0)),
                  pl.BlockSpec((1, K), lambda i: (0, 0)),
                  pl.BlockSpec((ED, K), lambda i: (0, 0)),
                  pl.BlockSpec((1, K), lambda i: (0, 0))],
        out_specs=[pl.BlockSpec((br, K), lambda i: (i, 0)),
                   pl.BlockSpec((br, K), lambda i: (i, 0))],
        out_shape=[jax.ShapeDtypeStruct((E, K), jnp.float32),
                   jax.ShapeDtypeStruct((E, K), jnp.float32)],
    )(ea, w1, b1.reshape(1, K), w2, b2.reshape(1, K))


def _node_body(sp_ref, xr_ref, bias_ref, g_ref, be_ref, o_ref):
    sp = sp_ref[0] + sp_ref[1]
    s = sp[:, 0:H]
    c = jnp.maximum(sp[:, H:H + 1], 1.0)
    z = xr_ref[...] + s / c + bias_ref[...]
    mu = jnp.mean(z, axis=0, keepdims=True)
    var = jnp.mean((z - mu) * (z - mu), axis=0, keepdims=True)
    o_ref[...] = jnp.maximum(
        (z - mu) * lax.rsqrt(var + 1e-5) * g_ref[...] + be_ref[...], 0.0)


def _node_update(sparts, xr, bias, g, be):
    return pl.pallas_call(
        _node_body,
        in_specs=[pl.BlockSpec((_NC, N, SW), lambda: (0, 0, 0)),
                  pl.BlockSpec((N, H), lambda: (0, 0)),
                  pl.BlockSpec((1, H), lambda: (0, 0)),
                  pl.BlockSpec((1, H), lambda: (0, 0)),
                  pl.BlockSpec((1, H), lambda: (0, 0))],
        out_specs=pl.BlockSpec((N, H), lambda: (0, 0)),
        out_shape=jax.ShapeDtypeStruct((N, H), jnp.float32),
    )(sparts, xr, bias.reshape(1, H), g.reshape(1, H),
      be.reshape(1, H))


def _pool_body(x2_ref, batch_ref, wl_ref, bl_ref, o_ref):
    x2 = x2_ref[...]
    b = batch_ref[...]
    gids = lax.broadcasted_iota(jnp.int32, (1, NG), 1)
    onehot = (b == gids).astype(jnp.float32)
    sums = lax.dot_general(onehot, x2, (((0,), (0,)), ((), ())),
                           preferred_element_type=jnp.float32)
    cnt = jnp.maximum(jnp.sum(onehot, axis=0), 1.0)
    pooled = sums / cnt[:, None]
    o_ref[...] = jnp.dot(pooled, wl_ref[...],
                         preferred_element_type=jnp.float32) + bl_ref[...]


def _pool_project(x2, batch, wl, bl):
    return pl.pallas_call(
        _pool_body,
        in_specs=[pl.BlockSpec((N, H), lambda: (0, 0)),
                  pl.BlockSpec((N, 1), lambda: (0, 0)),
                  pl.BlockSpec((H, OUT), lambda: (0, 0)),
                  pl.BlockSpec((1, OUT), lambda: (0, 0))],
        out_specs=pl.BlockSpec((NG, OUT), lambda: (0, 0)),
        out_shape=jax.ShapeDtypeStruct((NG, OUT), jnp.float32),
    )(x2, batch.reshape(N, 1), wl, bl.reshape(1, OUT))


SW = 128


def _sc_body(pcat_hbm, h_hbm, src_hbm, dst_hbm, zeros_hbm,
             s_out,
             srcA, srcB, dst_v, rowsA, rowsB, h_v, msg_v, s_sh,
             semA, semB):
    cid = lax.axis_index("c")
    sid = lax.axis_index("s")
    wid = sid * _NC + cid

    @pl.when(sid == 0)
    def _():
        pltpu.sync_copy(zeros_hbm, s_sh)
    plsc.subcore_barrier()

    zero16 = jnp.zeros((16,), jnp.float32)
    cnt16 = jnp.where(lax.iota(jnp.int32, 16) == 0, 1.0, 0.0)

    def _init_rows(b, carry):
        msg_v[b, pl.ds(16, 16)] = cnt16
        for j in range(2, 8):
            msg_v[b, pl.ds(j * 16, 16)] = zero16
        return carry
    lax.fori_loop(0, _B, _init_rows, 0)

    base0 = wid * _EW
    _BA = 24
    _BB = 16

    def _compute(rows_s, nb, off):
        def _edge(b, c2):
            ha = h_v[b + off, pl.ds(0, 16)]
            hb = h_v[b + off, pl.ds(16, 16)]
            acc0 = rows_s[b, pl.ds(512, 16)]
            acc1 = ha[0] * rows_s[b, pl.ds(0, 16)]
            for kk in range(1, 16):
                r = rows_s[b, pl.ds(kk * 16, 16)]
                if kk % 2 == 0:
                    acc1 = acc1 + ha[kk] * r
                else:
                    acc0 = acc0 + ha[kk] * r
            for kk in range(16):
                r = rows_s[b, pl.ds((16 + kk) * 16, 16)]
                if kk % 2 == 0:
                    acc0 = acc0 + hb[kk] * r
                else:
                    acc1 = acc1 + hb[kk] * r
            msg_v[b + off, pl.ds(0, 16)] = acc0 + acc1
            return c2
        lax.fori_loop(0, nb, _edge, 0)

    pltpu.sync_copy(src_hbm.at[pl.ds(base0, _BA)], srcA)
    pltpu.async_copy(pcat_hbm.at[srcA], rowsA, semA)

    def _iter(j, carry):
        base = base0 + j * _B
        pltpu.sync_copy(src_hbm.at[pl.ds(base + _BA, _BB)], srcB)
        pltpu.async_copy(pcat_hbm.at[srcB], rowsB, semB)
        pltpu.sync_copy(dst_hbm.at[pl.ds(base, _B)], dst_v)
        pltpu.sync_copy(h_hbm.at[pl.ds(base, _B)], h_v)

        pltpu.make_async_copy(pcat_hbm.at[srcA], rowsA, semA).wait()
        _compute(rowsA, _BA, 0)

        @pl.when(j < _ITERS - 1)
        def _():
            nbase = base0 + (j + 1) * _B
            pltpu.sync_copy(src_hbm.at[pl.ds(nbase, _BA)], srcA)
            pltpu.async_copy(pcat_hbm.at[srcA], rowsA, semA)

        pltpu.make_async_copy(pcat_hbm.at[srcB], rowsB, semB).wait()
        _compute(rowsB, _BB, _BA)

        pltpu.sync_copy(msg_v, s_sh.at[dst_v], add=True)
        return carry
    lax.fori_loop(0, _ITERS, _iter, 0)

    plsc.subcore_barrier()

    @pl.when(sid == 0)
    def _():
        pltpu.sync_copy(s_sh, s_out.at[cid])


_sc_edge = pl.kernel(
    _sc_body,
    out_type=jax.ShapeDtypeStruct((_NC, N, SW), jnp.float32),
    mesh=plsc.VectorSubcoreMesh(core_axis_name="c", subcore_axis_name="s"),
    scratch_types=[
        pltpu.VMEM((24,), jnp.int32),
        pltpu.VMEM((16,), jnp.int32),
        pltpu.VMEM((_B,), jnp.int32),
        pltpu.VMEM((24, CW), jnp.float32),
        pltpu.VMEM((16, CW), jnp.float32),
        pltpu.VMEM((_B, K), jnp.float32),
        pltpu.VMEM((_B, SW), jnp.float32),
        pltpu.VMEM_SHARED((N, SW), jnp.float32),
        pltpu.SemaphoreType.DMA,
        pltpu.SemaphoreType.DMA,
    ],
)


def kernel(x, edge_index, edge_attr, batch,
           W_e1a, b_e1a, W_e1b, b_e1b, root1, bias1, g1, be1,
           W_e2a, b_e2a, W_e2b, b_e2b, root2, bias2, g2, be2, Wl, bl):
    src = edge_index[0]
    dst = edge_index[1]
    zeros = jnp.zeros((N, SW), jnp.float32)

    a1 = W_e1b.reshape(K, IN, H).transpose(1, 0, 2).reshape(IN, K * H)
    acat1 = jnp.concatenate(
        [a1, b_e1b.reshape(IN, H), root1, jnp.zeros((IN, 96), jnp.float32)],
        axis=1)
    a2 = W_e2b.reshape(K, H, H).transpose(1, 0, 2).reshape(H, K * H)
    acat2 = jnp.concatenate(
        [a2, b_e2b.reshape(H, H), root2, jnp.zeros((H, 96), jnp.float32)],
        axis=1)

    h1, h2 = _edge_mlps(edge_attr, W_e1a, b_e1a, W_e2a, b_e2a)

    full1 = _matmul(x, acat1, 1000)
    s1 = _sc_edge(full1, h1, src, dst, zeros)
    x1 = _node_update(s1, full1[:, 528:544], bias1, g1, be1)

    full2 = _matmul(x1, acat2, 1000)
    s2 = _sc_edge(full2, h2, src, dst, zeros)
    x2 = _node_update(s2, full2[:, 528:544], bias2, g2, be2)

    return _pool_project(x2, batch, Wl, bl)


# speedup vs baseline: 2.2355x; 1.0620x over previous
import functools

import jax
import jax.numpy as jnp
from jax import lax
from jax.experimental import pallas as pl
from jax.experimental.pallas import tpu as pltpu
from jax.experimental.pallas import tpu_sc as plsc

N = 10000
E = 160000
IN = 128
ED = 16
H = 16
OUT = 128
NG = 64
K = 32
CW = 640

_info = plsc.get_sparse_core_info()
_NC = _info.num_cores
_NS = _info.num_subcores
_NW = _NC * _NS
_B = 40
_EW = E // _NW
_ITERS = _EW // _B


def _mm_body(a_ref, w_ref, o_ref):
    o_ref[...] = jnp.dot(a_ref[...], w_ref[...],
                         preferred_element_type=jnp.float32)


def _matmul(a, w, block_rows):
    m, k = a.shape
    _, c = w.shape
    return pl.pallas_call(
        _mm_body,
        grid=(m // block_rows,),
        in_specs=[pl.BlockSpec((block_rows, k), lambda i: (i, 0)),
                  pl.BlockSpec((k, c), lambda i: (0, 0))],
        out_specs=pl.BlockSpec((block_rows, c), lambda i: (i, 0)),
        out_shape=jax.ShapeDtypeStruct((m, c), jnp.float32),
    )(a, w)


def _h_body(ea_ref, w1_ref, b1_ref, w2_ref, b2_ref, h1_ref, h2_ref):
    ea = ea_ref[...]
    h1_ref[...] = jnp.maximum(
        jnp.dot(ea, w1_ref[...], preferred_element_type=jnp.float32)
        + b1_ref[...], 0.0)
    h2_ref[...] = jnp.maximum(
        jnp.dot(ea, w2_ref[...], preferred_element_type=jnp.float32)
        + b2_ref[...], 0.0)


def _edge_mlps(ea, w1, b1, w2, b2):
    br = 4000
    return pl.pallas_call(
        _h_body,
        grid=(E // br,),
        in_specs=[pl.BlockSpec((br, ED), lambda i: (i, 0)),
                  pl.BlockSpec((ED, K), lambda i: (0, 0)),
                  pl.BlockSpec((1, K), lambda i: (0, 0)),
                  pl.BlockSpec((ED, K), lambda i: (0, 0)),
                  pl.BlockSpec((1, K), lambda i: (0, 0))],
        out_specs=[pl.BlockSpec((br, K), lambda i: (i, 0)),
                   pl.BlockSpec((br, K), lambda i: (i, 0))],
        out_shape=[jax.ShapeDtypeStruct((E, K), jnp.float32),
                   jax.ShapeDtypeStruct((E, K), jnp.float32)],
    )(ea, w1, b1.reshape(1, K), w2, b2.reshape(1, K))


def _node_body(sp_ref, xr_ref, bias_ref, g_ref, be_ref, o_ref):
    sp = sp_ref[0] + sp_ref[1]
    s = sp[:, 0:H]
    c = jnp.maximum(sp[:, H:H + 1], 1.0)
    z = xr_ref[...] + s / c + bias_ref[...]
    mu = jnp.mean(z, axis=0, keepdims=True)
    var = jnp.mean((z - mu) * (z - mu), axis=0, keepdims=True)
    o_ref[...] = jnp.maximum(
        (z - mu) * lax.rsqrt(var + 1e-5) * g_ref[...] + be_ref[...], 0.0)


def _node_update(sparts, xr, bias, g, be):
    return pl.pallas_call(
        _node_body,
        in_specs=[pl.BlockSpec((_NC, N, SW), lambda: (0, 0, 0)),
                  pl.BlockSpec((N, H), lambda: (0, 0)),
                  pl.BlockSpec((1, H), lambda: (0, 0)),
                  pl.BlockSpec((1, H), lambda: (0, 0)),
                  pl.BlockSpec((1, H), lambda: (0, 0))],
        out_specs=pl.BlockSpec((N, H), lambda: (0, 0)),
        out_shape=jax.ShapeDtypeStruct((N, H), jnp.float32),
    )(sparts, xr, bias.reshape(1, H), g.reshape(1, H),
      be.reshape(1, H))


def _pool_body(x2_ref, batch_ref, wl_ref, bl_ref, o_ref):
    x2 = x2_ref[...]
    b = batch_ref[...]
    gids = lax.broadcasted_iota(jnp.int32, (1, NG), 1)
    onehot = (b == gids).astype(jnp.float32)
    sums = lax.dot_general(onehot, x2, (((0,), (0,)), ((), ())),
                           preferred_element_type=jnp.float32)
    cnt = jnp.maximum(jnp.sum(onehot, axis=0), 1.0)
    pooled = sums / cnt[:, None]
    o_ref[...] = jnp.dot(pooled, wl_ref[...],
                         preferred_element_type=jnp.float32) + bl_ref[...]


def _pool_project(x2, batch, wl, bl):
    return pl.pallas_call(
        _pool_body,
        in_specs=[pl.BlockSpec((N, H), lambda: (0, 0)),
                  pl.BlockSpec((N, 1), lambda: (0, 0)),
                  pl.BlockSpec((H, OUT), lambda: (0, 0)),
                  pl.BlockSpec((1, OUT), lambda: (0, 0))],
        out_specs=pl.BlockSpec((NG, OUT), lambda: (0, 0)),
        out_shape=jax.ShapeDtypeStruct((NG, OUT), jnp.float32),
    )(x2, batch.reshape(N, 1), wl, bl.reshape(1, OUT))


SW = 128


def _sc_body(pcat_hbm, h_hbm, src_hbm, dst_hbm, zeros_hbm,
             s_out,
             srcA, srcB, dst_v, rowsA, rowsB, h_v, msg_v, s_sh,
             semA, semB):
    cid = lax.axis_index("c")
    sid = lax.axis_index("s")
    wid = sid * _NC + cid

    @pl.when(sid == 0)
    def _():
        pltpu.sync_copy(zeros_hbm, s_sh)
    plsc.subcore_barrier()

    zero16 = jnp.zeros((16,), jnp.float32)
    cnt16 = jnp.where(lax.iota(jnp.int32, 16) == 0, 1.0, 0.0)

    def _init_rows(b, carry):
        msg_v[b, pl.ds(16, 16)] = cnt16
        for j in range(2, 8):
            msg_v[b, pl.ds(j * 16, 16)] = zero16
        return carry
    lax.fori_loop(0, _B, _init_rows, 0)

    base0 = wid * _EW
    _BA = 24
    _BB = 16

    def _compute(rows_s, nb, off):
        @plsc.parallel_loop(0, nb, unroll=4)
        def _edge(b):
            ha = h_v[b + off, pl.ds(0, 16)]
            hb = h_v[b + off, pl.ds(16, 16)]
            acc0 = rows_s[b, pl.ds(512, 16)]
            acc1 = ha[0] * rows_s[b, pl.ds(0, 16)]
            for kk in range(1, 16):
                r = rows_s[b, pl.ds(kk * 16, 16)]
                if kk % 2 == 0:
                    acc1 = acc1 + ha[kk] * r
                else:
                    acc0 = acc0 + ha[kk] * r
            for kk in range(16):
                r = rows_s[b, pl.ds((16 + kk) * 16, 16)]
                if kk % 2 == 0:
                    acc0 = acc0 + hb[kk] * r
                else:
                    acc1 = acc1 + hb[kk] * r
            msg_v[b + off, pl.ds(0, 16)] = acc0 + acc1

    pltpu.sync_copy(src_hbm.at[pl.ds(base0, _BA)], srcA)
    pltpu.async_copy(pcat_hbm.at[srcA], rowsA, semA)

    def _iter(j, carry):
        base = base0 + j * _B
        pltpu.sync_copy(src_hbm.at[pl.ds(base + _BA, _BB)], srcB)
        pltpu.async_copy(pcat_hbm.at[srcB], rowsB, semB)
        pltpu.sync_copy(dst_hbm.at[pl.ds(base, _B)], dst_v)
        pltpu.sync_copy(h_hbm.at[pl.ds(base, _B)], h_v)

        pltpu.make_async_copy(pcat_hbm.at[srcA], rowsA, semA).wait()
        _compute(rowsA, _BA, 0)

        @pl.when(j < _ITERS - 1)
        def _():
            nbase = base0 + (j + 1) * _B
            pltpu.sync_copy(src_hbm.at[pl.ds(nbase, _BA)], srcA)
            pltpu.async_copy(pcat_hbm.at[srcA], rowsA, semA)

        pltpu.make_async_copy(pcat_hbm.at[srcB], rowsB, semB).wait()
        _compute(rowsB, _BB, _BA)

        pltpu.sync_copy(msg_v, s_sh.at[dst_v], add=True)
        return carry
    lax.fori_loop(0, _ITERS, _iter, 0)

    plsc.subcore_barrier()

    @pl.when(sid == 0)
    def _():
        pltpu.sync_copy(s_sh, s_out.at[cid])


_sc_edge = pl.kernel(
    _sc_body,
    out_type=jax.ShapeDtypeStruct((_NC, N, SW), jnp.float32),
    mesh=plsc.VectorSubcoreMesh(core_axis_name="c", subcore_axis_name="s"),
    scratch_types=[
        pltpu.VMEM((24,), jnp.int32),
        pltpu.VMEM((16,), jnp.int32),
        pltpu.VMEM((_B,), jnp.int32),
        pltpu.VMEM((24, CW), jnp.float32),
        pltpu.VMEM((16, CW), jnp.float32),
        pltpu.VMEM((_B, K), jnp.float32),
        pltpu.VMEM((_B, SW), jnp.float32),
        pltpu.VMEM_SHARED((N, SW), jnp.float32),
        pltpu.SemaphoreType.DMA,
        pltpu.SemaphoreType.DMA,
    ],
)


def kernel(x, edge_index, edge_attr, batch,
           W_e1a, b_e1a, W_e1b, b_e1b, root1, bias1, g1, be1,
           W_e2a, b_e2a, W_e2b, b_e2b, root2, bias2, g2, be2, Wl, bl):
    src = edge_index[0]
    dst = edge_index[1]
    zeros = jnp.zeros((N, SW), jnp.float32)

    a1 = W_e1b.reshape(K, IN, H).transpose(1, 0, 2).reshape(IN, K * H)
    acat1 = jnp.concatenate(
        [a1, b_e1b.reshape(IN, H), root1, jnp.zeros((IN, 96), jnp.float32)],
        axis=1)
    a2 = W_e2b.reshape(K, H, H).transpose(1, 0, 2).reshape(H, K * H)
    acat2 = jnp.concatenate(
        [a2, b_e2b.reshape(H, H), root2, jnp.zeros((H, 96), jnp.float32)],
        axis=1)

    h1, h2 = _edge_mlps(edge_attr, W_e1a, b_e1a, W_e2a, b_e2a)

    full1 = _matmul(x, acat1, 1000)
    s1 = _sc_edge(full1, h1, src, dst, zeros)
    x1 = _node_update(s1, full1[:, 528:544], bias1, g1, be1)

    full2 = _matmul(x1, acat2, 1000)
    s2 = _sc_edge(full2, h2, src, dst, zeros)
    x2 = _node_update(s2, full2[:, 528:544], bias2, g2, be2)

    return _pool_project(x2, batch, Wl, bl)
